# baseline (speedup 1.0000x reference)
import functools

import jax
import jax.numpy as jnp
import numpy as np
from jax import lax
from jax.experimental import pallas as pl
from jax.experimental.pallas import tpu as pltpu
from jax.experimental.pallas import tpu_sc as plsc

ROWS, COLS = 2048, 8192
N = ROWS * COLS
NW = 32
PER_W = N // NW
ROWS_W = ROWS // NW
CHUNK_R, CHUNK_C = 8, 4096
CHUNK = CHUNK_R * CHUNK_C
NCHUNK = PER_W // CHUNK
UNROLL = 16
NROW = 130
HSIZE = NROW * 16
LUTN = 1280

_S = np.log10(2.0) * 63.0 / 13.0
C1 = float(np.float32(_S / 2 ** 23))
KC = float(np.float32(-127.0 * _S + 7.0 * 63.0 / 13.0 + 513.0 - 1e-3))

_RR = np.arange(360, 737)
_CC = np.clip(_RR - 512, 0, 64)

_RLUT_NP = np.zeros(LUTN, np.int32)
_RLUT_NP[360:737] = (65 + _CC) << 4
_RLUT_NP[872:1249] = (64 - _CC) << 4


def _edges():
    return jnp.concatenate(
        [-jnp.logspace(6, -7, 64), jnp.array([0.0]), jnp.logspace(-7, 6, 64)]
    )


def _blut(edges):
    bpos = edges[65:129]
    bpos = jnp.concatenate([bpos[:63], jnp.nextafter(bpos[63:64], jnp.inf)])
    bneg = jnp.nextafter(-edges[0:64][::-1], jnp.inf)
    z = lambda n: jnp.zeros((n,), jnp.float32)
    inf161 = jnp.full((161,), jnp.inf, jnp.float32)
    rep = lambda v: jnp.broadcast_to(v, (153,))
    return jnp.concatenate([
        z(360), rep(bpos[0]), bpos[1:64], inf161,
        z(135), rep(bneg[0]), bneg[1:64], inf161,
        z(31),
    ])


def _sc_body(data_hbm, blut_hbm, rlut_hbm, hist_out,
             buf0, buf1, blut_v, rlut_v, hist_v, sem0, sem1):
    wid = lax.axis_index("s") * 2 + lax.axis_index("c")
    base_row = wid * ROWS_W

    pltpu.sync_copy(blut_hbm, blut_v)
    pltpu.sync_copy(rlut_hbm, rlut_v)
    z16 = jnp.zeros((16,), jnp.int32)
    for r in range(NROW):
        hist_v[pl.ds(r * 16, 16)] = z16

    lane = lax.iota(jnp.int32, 16)
    ones = jnp.ones((16,), jnp.int32)

    bufs = (buf0, buf1)
    sems = (sem0, sem1)

    def src(c):
        return data_hbm.at[pl.ds(base_row + (c // 2) * CHUNK_R, CHUNK_R),
                           pl.ds((c % 2) * CHUNK_C, CHUNK_C)]

    pltpu.make_async_copy(src(0), bufs[0], sems[0]).start()

    for c in range(NCHUNK):
        cur, csem = bufs[c % 2], sems[c % 2]
        if c + 1 < NCHUNK:
            nxt, nsem = bufs[(c + 1) % 2], sems[(c + 1) % 2]
            pltpu.make_async_copy(src(c + 1), nxt, nsem).start()
        pltpu.make_async_copy(src(c), cur, csem).wait()

        @plsc.parallel_loop(0, CHUNK // 16, unroll=UNROLL)
        def step(v):
            x = cur[v // (CHUNK_C // 16), pl.ds((v % (CHUNK_C // 16)) * 16, 16)]
            xi = plsc.bitcast(x, jnp.int32)
            ai = xi & 0x7FFFFFFF
            af = plsc.bitcast(ai, jnp.float32)
            fi = ai.astype(jnp.float32)
            kt = fi * C1 + KC
            ri = kt.astype(jnp.int32)
            sgn = lax.shift_right_logical(xi, 31)
            idx = ri + (sgn << 9)
            b = plsc.load_gather(blut_v, [idx])
            rb = plsc.load_gather(rlut_v, [idx])
            d16 = 16 - (sgn << 5)
            slot = rb + jnp.where(af >= b, d16, 0) + lane
            plsc.addupdate_scatter(hist_v, [slot], ones)

    pltpu.sync_copy(hist_v, hist_out.at[wid])


_sc_hist = functools.partial(
    pl.kernel,
    out_type=jax.ShapeDtypeStruct((NW, HSIZE), jnp.int32),
    mesh=plsc.VectorSubcoreMesh(core_axis_name="c", subcore_axis_name="s"),
    compiler_params=pltpu.CompilerParams(needs_layout_passes=False),
    scratch_types=[
        pltpu.VMEM((CHUNK_R, CHUNK_C), jnp.float32),
        pltpu.VMEM((CHUNK_R, CHUNK_C), jnp.float32),
        pltpu.VMEM((LUTN,), jnp.float32),
        pltpu.VMEM((LUTN,), jnp.int32),
        pltpu.VMEM((HSIZE,), jnp.int32),
        pltpu.SemaphoreType.DMA,
        pltpu.SemaphoreType.DMA,
    ],
)(_sc_body)

TC_BLOCK_ROWS = 256
TC_GRID = ROWS // TC_BLOCK_ROWS


def _tc_stats_body(data_ref, stats_ref):
    pi = pl.program_id(0)

    @pl.when(pi == 0)
    def _init():
        stats_ref[0] = jnp.inf
        stats_ref[1] = -jnp.inf
        stats_ref[2] = 0.0
        stats_ref[3] = 0.0

    block = data_ref[...]
    stats_ref[0] = jnp.minimum(stats_ref[0], jnp.min(block))
    stats_ref[1] = jnp.maximum(stats_ref[1], jnp.max(block))
    stats_ref[2] += jnp.sum(block)
    stats_ref[3] += jnp.sum(block * block)


def _tc_stats(data):
    return pl.pallas_call(
        _tc_stats_body,
        grid=(TC_GRID,),
        in_specs=[pl.BlockSpec((TC_BLOCK_ROWS, COLS), lambda i: (i, 0))],
        out_specs=pl.BlockSpec(memory_space=pltpu.SMEM),
        out_shape=jax.ShapeDtypeStruct((4,), jnp.float32),
    )(data)


def kernel(data):
    edges = _edges()
    hist_parts = _sc_hist(data, _blut(edges), jnp.asarray(_RLUT_NP))
    stats = _tc_stats(data)
    hist = hist_parts.reshape(NW, NROW, 16).sum((0, 2))
    counts = hist[1:129].astype(jnp.float32)
    num = jnp.asarray(data.size, jnp.int32)
    return (stats[0], stats[1], num, stats[2], stats[3], edges, counts)


# speedup vs baseline: 1.0879x; 1.0879x over previous
import functools

import jax
import jax.numpy as jnp
import numpy as np
from jax import lax
from jax.experimental import pallas as pl
from jax.experimental.pallas import tpu as pltpu
from jax.experimental.pallas import tpu_sc as plsc

ROWS, COLS = 2048, 8192
N = ROWS * COLS
NW = 32
PER_W = N // NW
ROWS_W = ROWS // NW
CHUNK_R, CHUNK_C = 8, 4096
CHUNK = CHUNK_R * CHUNK_C
NCHUNK = PER_W // CHUNK
UNROLL = 4
NROW = 130
HSIZE = NROW * 16
LUTN = 1280

_S = np.log10(2.0) * 63.0 / 13.0
C1 = float(np.float32(_S / 2 ** 23))
KC = float(np.float32(-127.0 * _S + 7.0 * 63.0 / 13.0 + 513.0 - 1e-3))

_RR = np.arange(360, 737)
_CC = np.clip(_RR - 512, 0, 64)

_RLUT_NP = np.zeros(LUTN, np.int32)
_RLUT_NP[360:737] = (65 + _CC) << 4
_RLUT_NP[872:1249] = (64 - _CC) << 4


def _edges():
    return jnp.concatenate(
        [-jnp.logspace(6, -7, 64), jnp.array([0.0]), jnp.logspace(-7, 6, 64)]
    )


def _blut(edges):
    bpos = edges[65:129]
    bpos = jnp.concatenate([bpos[:63], jnp.nextafter(bpos[63:64], jnp.inf)])
    bneg = jnp.nextafter(-edges[0:64][::-1], jnp.inf)
    z = lambda n: jnp.zeros((n,), jnp.float32)
    inf161 = jnp.full((161,), jnp.inf, jnp.float32)
    rep = lambda v: jnp.broadcast_to(v, (153,))
    return jnp.concatenate([
        z(360), rep(bpos[0]), bpos[1:64], inf161,
        z(135), rep(bneg[0]), bneg[1:64], inf161,
        z(31),
    ])


def _sc_body(data_hbm, blut_hbm, rlut_hbm, hist_out,
             buf0, buf1, blut_v, rlut_v, hist_v, sem0, sem1):
    wid = lax.axis_index("s") * 2 + lax.axis_index("c")
    base_row = wid * ROWS_W

    pltpu.sync_copy(blut_hbm, blut_v)
    pltpu.sync_copy(rlut_hbm, rlut_v)
    z16 = jnp.zeros((16,), jnp.int32)
    for r in range(NROW):
        hist_v[pl.ds(r * 16, 16)] = z16

    lane = lax.iota(jnp.int32, 16)
    ones = jnp.ones((16,), jnp.int32)

    bufs = (buf0, buf1)
    sems = (sem0, sem1)

    def src(c):
        return data_hbm.at[pl.ds(base_row + (c // 2) * CHUNK_R, CHUNK_R),
                           pl.ds((c % 2) * CHUNK_C, CHUNK_C)]

    pltpu.make_async_copy(src(0), bufs[0], sems[0]).start()

    for c in range(NCHUNK):
        cur, csem = bufs[c % 2], sems[c % 2]
        if c + 1 < NCHUNK:
            nxt, nsem = bufs[(c + 1) % 2], sems[(c + 1) % 2]
            pltpu.make_async_copy(src(c + 1), nxt, nsem).start()
        pltpu.make_async_copy(src(c), cur, csem).wait()

        @plsc.parallel_loop(0, CHUNK // 16, unroll=UNROLL)
        def step(v):
            x = cur[v // (CHUNK_C // 16), pl.ds((v % (CHUNK_C // 16)) * 16, 16)]
            xi = plsc.bitcast(x, jnp.int32)
            ai = xi & 0x7FFFFFFF
            af = plsc.bitcast(ai, jnp.float32)
            fi = ai.astype(jnp.float32)
            kt = fi * C1 + KC
            ri = kt.astype(jnp.int32)
            sgn = lax.shift_right_logical(xi, 31)
            idx = ri + (sgn << 9)
            b = plsc.load_gather(blut_v, [idx])
            rb = plsc.load_gather(rlut_v, [idx])
            d16 = 16 - (sgn << 5)
            slot = rb + jnp.where(af >= b, d16, 0) + lane
            plsc.addupdate_scatter(hist_v, [slot], ones)

    pltpu.sync_copy(hist_v, hist_out.at[wid])


_sc_hist = functools.partial(
    pl.kernel,
    out_type=jax.ShapeDtypeStruct((NW, HSIZE), jnp.int32),
    mesh=plsc.VectorSubcoreMesh(core_axis_name="c", subcore_axis_name="s"),
    compiler_params=pltpu.CompilerParams(needs_layout_passes=False),
    scratch_types=[
        pltpu.VMEM((CHUNK_R, CHUNK_C), jnp.float32),
        pltpu.VMEM((CHUNK_R, CHUNK_C), jnp.float32),
        pltpu.VMEM((LUTN,), jnp.float32),
        pltpu.VMEM((LUTN,), jnp.int32),
        pltpu.VMEM((HSIZE,), jnp.int32),
        pltpu.SemaphoreType.DMA,
        pltpu.SemaphoreType.DMA,
    ],
)(_sc_body)

TC_BLOCK_ROWS = 256
TC_GRID = ROWS // TC_BLOCK_ROWS


def _tc_stats_body(data_ref, stats_ref):
    pi = pl.program_id(0)

    @pl.when(pi == 0)
    def _init():
        stats_ref[0] = jnp.inf
        stats_ref[1] = -jnp.inf
        stats_ref[2] = 0.0
        stats_ref[3] = 0.0

    block = data_ref[...]
    stats_ref[0] = jnp.minimum(stats_ref[0], jnp.min(block))
    stats_ref[1] = jnp.maximum(stats_ref[1], jnp.max(block))
    stats_ref[2] += jnp.sum(block)
    stats_ref[3] += jnp.sum(block * block)


def _tc_stats(data):
    return pl.pallas_call(
        _tc_stats_body,
        grid=(TC_GRID,),
        in_specs=[pl.BlockSpec((TC_BLOCK_ROWS, COLS), lambda i: (i, 0))],
        out_specs=pl.BlockSpec(memory_space=pltpu.SMEM),
        out_shape=jax.ShapeDtypeStruct((4,), jnp.float32),
    )(data)


def kernel(data):
    edges = _edges()
    hist_parts = _sc_hist(data, _blut(edges), jnp.asarray(_RLUT_NP))
    stats = _tc_stats(data)
    hist = hist_parts.reshape(NW, NROW, 16).sum((0, 2))
    counts = hist[1:129].astype(jnp.float32)
    num = jnp.asarray(data.size, jnp.int32)
    return (stats[0], stats[1], num, stats[2], stats[3], edges, counts)
